# baseline (speedup 1.0000x reference)
import jax
import jax.numpy as jnp
from jax.experimental import pallas as pl
from jax.experimental.pallas import tpu as pltpu

_BN = 512
_MP = 4


def _body(x_hbm, w_ref, b_ref, o_ref, xv_ref, sems):
    j = pl.program_id(0)
    m = pl.program_id(1)
    mh = x_hbm.shape[0] // _MP

    @pl.when(jnp.logical_and(j == 0, m == 0))
    def _start_copies():
        for c in range(_MP):
            pltpu.make_async_copy(
                x_hbm.at[c * mh:(c + 1) * mh, :],
                xv_ref.at[c * mh:(c + 1) * mh, :],
                sems.at[c],
            ).start()

    for c in range(_MP):
        @pl.when(jnp.logical_and(j == 0, m == c))
        def _wait_chunk(c=c):
            pltpu.make_async_copy(
                x_hbm.at[c * mh:(c + 1) * mh, :],
                xv_ref.at[c * mh:(c + 1) * mh, :],
                sems.at[c],
            ).wait()

    o_ref[...] = jax.lax.dot_general(
        xv_ref[pl.ds(m * mh, mh), :], w_ref[...],
        (((1,), (1,)), ((), ())),
        preferred_element_type=jnp.float32,
    ) + b_ref[...]


def kernel(x, weight, bias):
    if x.ndim == 4:
        x = x.reshape(x.shape[0], -1)
    M, K = x.shape
    N = weight.shape[0]
    bias2 = bias.reshape(1, N)
    mh = M // _MP
    grid = (N // _BN, _MP)
    return pl.pallas_call(
        _body,
        grid=grid,
        in_specs=[
            pl.BlockSpec(memory_space=pl.ANY),
            pl.BlockSpec((_BN, K), lambda j, m: (j, 0)),
            pl.BlockSpec((1, _BN), lambda j, m: (0, j)),
        ],
        out_specs=pl.BlockSpec((mh, _BN), lambda j, m: (m, j)),
        out_shape=jax.ShapeDtypeStruct((M, N), jnp.float32),
        scratch_shapes=[
            pltpu.VMEM((M, K), jnp.float32),
            pltpu.SemaphoreType.DMA((_MP,)),
        ],
        compiler_params=pltpu.CompilerParams(
            dimension_semantics=("arbitrary", "arbitrary"),
            vmem_limit_bytes=64 * 1024 * 1024,
        ),
    )(x, weight, bias2)


# speedup vs baseline: 1.1315x; 1.1315x over previous
import jax
import jax.numpy as jnp
from jax.experimental import pallas as pl
from jax.experimental.pallas import tpu as pltpu

_BN = 512


def _body(x_ref, w_ref, b_ref, o_ref):
    o_ref[...] = jax.lax.dot_general(
        x_ref[...], w_ref[...],
        (((1,), (1,)), ((), ())),
        preferred_element_type=jnp.float32,
    ) + b_ref[...]


def kernel(x, weight, bias):
    if x.ndim == 4:
        x = x.reshape(x.shape[0], -1)
    M, K = x.shape
    N = weight.shape[0]
    bias2 = bias.reshape(1, N)
    grid = (N // _BN,)
    return pl.pallas_call(
        _body,
        grid=grid,
        in_specs=[
            pl.BlockSpec((M, K), lambda j: (0, 0)),
            pl.BlockSpec((_BN, K), lambda j: (j, 0)),
            pl.BlockSpec((1, _BN), lambda j: (0, j)),
        ],
        out_specs=pl.BlockSpec((M, _BN), lambda j: (0, j)),
        out_shape=jax.ShapeDtypeStruct((M, N), jnp.float32),
        compiler_params=pltpu.CompilerParams(
            dimension_semantics=("arbitrary",),
            vmem_limit_bytes=64 * 1024 * 1024,
        ),
    )(x, weight, bias2)
